# baseline (device time: 160830 ns/iter reference)
import jax
import jax.numpy as jnp
from jax import lax
from jax.experimental import pallas as pl
from jax.experimental.pallas import tpu as pltpu

N_DEV = 32
E_PER = 2


def kernel(x, router_W, route_idx, expert_W):
    n_tok, d = x.shape
    n_exp = router_W.shape[1]
    h = expert_W.shape[2]

    def body(x_ref, rw_ref, idx_ref, ew_ref, out_ref,
             comm_ref, send_sems, recv_sems):
        my_pos = lax.axis_index("i")
        left = lax.rem(my_pos - 1 + N_DEV, N_DEV)
        right = lax.rem(my_pos + 1, N_DEV)

        barrier_sem = pltpu.get_barrier_semaphore()
        for nbr in (left, right):
            pl.semaphore_signal(
                barrier_sem, inc=1,
                device_id=(nbr,), device_id_type=pl.DeviceIdType.MESH,
            )
        pl.semaphore_wait(barrier_sem, 2)

        xv = x_ref[:, :]
        scores = jnp.dot(xv, rw_ref[:, :], preferred_element_type=jnp.float32)
        s_max = jnp.max(scores, axis=-1, keepdims=True)
        p = jnp.exp(scores - s_max)
        probs = p / jnp.sum(p, axis=-1, keepdims=True)

        e0 = idx_ref[:, 0:1]
        e1 = idx_ref[:, 1:2]
        iota = lax.broadcasted_iota(jnp.int32, (n_tok, n_exp), 1)
        g0 = jnp.sum(jnp.where(iota == e0, probs, 0.0), axis=1, keepdims=True)
        g1 = jnp.sum(jnp.where(iota == e1, probs, 0.0), axis=1, keepdims=True)
        gs = g0 + g1
        w0 = g0 / gs
        w1 = g1 / gs

        def contrib(pair_ref, origin):
            acc = None
            for le in range(E_PER):
                ge = origin * E_PER + le
                m = (w0 * (e0 == ge).astype(jnp.float32)
                     + w1 * (e1 == ge).astype(jnp.float32))
                y = jnp.dot(xv, pair_ref[le],
                            preferred_element_type=jnp.float32)
                t = y * m
                acc = t if acc is None else acc + t
            return acc

        out_ref[:, :] = contrib(ew_ref, my_pos)
        comm_ref[0] = ew_ref[...]

        for hop in range(N_DEV - 1):
            send_slot = hop % 2
            recv_slot = (hop + 1) % 2
            rdma = pltpu.make_async_remote_copy(
                src_ref=comm_ref.at[send_slot],
                dst_ref=comm_ref.at[recv_slot],
                send_sem=send_sems.at[send_slot],
                recv_sem=recv_sems.at[recv_slot],
                device_id=(right,),
                device_id_type=pl.DeviceIdType.MESH,
            )
            rdma.start()
            rdma.wait()
            origin = lax.rem(my_pos - hop - 1 + N_DEV, N_DEV)
            out_ref[:, :] = out_ref[:, :] + contrib(comm_ref.at[recv_slot],
                                                    origin)

    return pl.pallas_call(
        body,
        out_shape=jax.ShapeDtypeStruct((n_tok, h), jnp.float32),
        in_specs=[pl.BlockSpec(memory_space=pltpu.VMEM)] * 4,
        out_specs=pl.BlockSpec(memory_space=pltpu.VMEM),
        scratch_shapes=[
            pltpu.VMEM((2, E_PER, d, h), jnp.float32),
            pltpu.SemaphoreType.DMA((2,)),
            pltpu.SemaphoreType.DMA((2,)),
        ],
        compiler_params=pltpu.CompilerParams(collective_id=0),
    )(x, router_W, route_idx, expert_W)


# device time: 98779 ns/iter; 1.6282x vs baseline; 1.6282x over previous
import jax
import jax.numpy as jnp
from jax import lax
from jax.experimental import pallas as pl
from jax.experimental.pallas import tpu as pltpu

N_DEV = 32
E_PER = 2
R_HOPS = N_DEV // 2
L_HOPS = N_DEV - 1 - R_HOPS


def kernel(x, router_W, route_idx, expert_W):
    n_tok, d = x.shape
    n_exp = router_W.shape[1]
    h = expert_W.shape[2]

    def body(x_ref, rw_ref, idx_ref, ew_ref, out_ref,
             bufR, bufL, sendR_sems, recvR_sems, sendL_sems, recvL_sems):
        my_pos = lax.axis_index("i")
        left = lax.rem(my_pos - 1 + N_DEV, N_DEV)
        right = lax.rem(my_pos + 1, N_DEV)

        barrier_sem = pltpu.get_barrier_semaphore()
        for nbr in (left, right):
            pl.semaphore_signal(
                barrier_sem, inc=1,
                device_id=(nbr,), device_id_type=pl.DeviceIdType.MESH,
            )
        pl.semaphore_wait(barrier_sem, 2)

        def send(src_ref, dst_ref, send_sem, recv_sem, dev):
            rdma = pltpu.make_async_remote_copy(
                src_ref=src_ref, dst_ref=dst_ref,
                send_sem=send_sem, recv_sem=recv_sem,
                device_id=(dev,), device_id_type=pl.DeviceIdType.MESH,
            )
            rdma.start()
            return rdma

        def recv_wait(dst_ref, recv_sem):
            rdma = pltpu.make_async_remote_copy(
                src_ref=dst_ref, dst_ref=dst_ref,
                send_sem=recv_sem, recv_sem=recv_sem,
                device_id=(left,), device_id_type=pl.DeviceIdType.MESH,
            )
            rdma.wait_recv()

        sends = [
            send(ew_ref, bufR.at[0], sendR_sems.at[0], recvR_sems.at[0], right),
            send(ew_ref, bufL.at[0], sendL_sems.at[0], recvL_sems.at[0], left),
        ]

        xv = x_ref[:, :]
        scores = jnp.dot(xv, rw_ref[:, :], preferred_element_type=jnp.float32)
        s_max = jnp.max(scores, axis=-1, keepdims=True)
        p = jnp.exp(scores - s_max)
        probs = p / jnp.sum(p, axis=-1, keepdims=True)

        e0 = idx_ref[:, 0:1]
        e1 = idx_ref[:, 1:2]
        iota = lax.broadcasted_iota(jnp.int32, (n_tok, n_exp), 1)
        g0 = jnp.sum(jnp.where(iota == e0, probs, 0.0), axis=1, keepdims=True)
        g1 = jnp.sum(jnp.where(iota == e1, probs, 0.0), axis=1, keepdims=True)
        gs = g0 + g1
        w0 = g0 / gs
        w1 = g1 / gs

        def contrib(pair_ref, origin):
            acc = None
            for le in range(E_PER):
                ge = origin * E_PER + le
                m = (w0 * (e0 == ge).astype(jnp.float32)
                     + w1 * (e1 == ge).astype(jnp.float32))
                y = jnp.dot(xv, pair_ref[le],
                            preferred_element_type=jnp.float32)
                t = y * m
                acc = t if acc is None else acc + t
            return acc

        out_ref[:, :] = contrib(ew_ref, my_pos)

        for hp in range(max(R_HOPS, L_HOPS)):
            if hp < R_HOPS:
                recv_wait(bufR.at[hp], recvR_sems.at[hp])
                if hp + 1 < R_HOPS:
                    sends.append(send(bufR.at[hp], bufR.at[hp + 1],
                                      sendR_sems.at[hp + 1],
                                      recvR_sems.at[hp + 1], right))
                origin = lax.rem(my_pos - hp - 1 + N_DEV, N_DEV)
                out_ref[:, :] = out_ref[:, :] + contrib(bufR.at[hp], origin)
            if hp < L_HOPS:
                recv_wait(bufL.at[hp], recvL_sems.at[hp])
                if hp + 1 < L_HOPS:
                    sends.append(send(bufL.at[hp], bufL.at[hp + 1],
                                      sendL_sems.at[hp + 1],
                                      recvL_sems.at[hp + 1], left))
                origin = lax.rem(my_pos + hp + 1, N_DEV)
                out_ref[:, :] = out_ref[:, :] + contrib(bufL.at[hp], origin)

        for s in sends:
            s.wait_send()

    return pl.pallas_call(
        body,
        out_shape=jax.ShapeDtypeStruct((n_tok, h), jnp.float32),
        in_specs=[pl.BlockSpec(memory_space=pltpu.VMEM)] * 4,
        out_specs=pl.BlockSpec(memory_space=pltpu.VMEM),
        scratch_shapes=[
            pltpu.VMEM((R_HOPS, E_PER, d, h), jnp.float32),
            pltpu.VMEM((L_HOPS, E_PER, d, h), jnp.float32),
            pltpu.SemaphoreType.DMA((R_HOPS,)),
            pltpu.SemaphoreType.DMA((R_HOPS,)),
            pltpu.SemaphoreType.DMA((L_HOPS,)),
            pltpu.SemaphoreType.DMA((L_HOPS,)),
        ],
        compiler_params=pltpu.CompilerParams(collective_id=0),
    )(x, router_W, route_idx, expert_W)


# device time: 72466 ns/iter; 2.2194x vs baseline; 1.3631x over previous
import jax
import jax.numpy as jnp
from jax import lax
from jax.experimental import pallas as pl
from jax.experimental.pallas import tpu as pltpu

N_DEV = 32
E_PER = 2
R_HOPS = N_DEV // 2
L_HOPS = N_DEV - 1 - R_HOPS


def kernel(x, router_W, route_idx, expert_W):
    n_tok, d = x.shape
    n_exp = router_W.shape[1]
    h = expert_W.shape[2]

    def body(x_ref, rw_ref, idx_ref, ew_ref, out_ref,
             own_buf, bufR, bufL,
             sendR_sems, recvR_sems, sendL_sems, recvL_sems):
        my_pos = lax.axis_index("i")
        left = lax.rem(my_pos - 1 + N_DEV, N_DEV)
        right = lax.rem(my_pos + 1, N_DEV)

        barrier_sem = pltpu.get_barrier_semaphore()
        for nbr in (left, right):
            pl.semaphore_signal(
                barrier_sem, inc=1,
                device_id=(nbr,), device_id_type=pl.DeviceIdType.MESH,
            )
        pl.semaphore_wait(barrier_sem, 2)

        def send(src_ref, dst_ref, send_sem, recv_sem, dev):
            rdma = pltpu.make_async_remote_copy(
                src_ref=src_ref, dst_ref=dst_ref,
                send_sem=send_sem, recv_sem=recv_sem,
                device_id=(dev,), device_id_type=pl.DeviceIdType.MESH,
            )
            rdma.start()
            return rdma

        def recv_wait(dst_ref, recv_sem):
            rdma = pltpu.make_async_remote_copy(
                src_ref=dst_ref, dst_ref=dst_ref,
                send_sem=recv_sem, recv_sem=recv_sem,
                device_id=(left,), device_id_type=pl.DeviceIdType.MESH,
            )
            rdma.wait_recv()

        own_buf[...] = ew_ref[...].astype(jnp.bfloat16)
        sends = [
            send(own_buf, bufR.at[0], sendR_sems.at[0], recvR_sems.at[0], right),
            send(own_buf, bufL.at[0], sendL_sems.at[0], recvL_sems.at[0], left),
        ]

        xv = x_ref[:, :]
        xv_bf = xv.astype(jnp.bfloat16)
        scores = jnp.dot(xv, rw_ref[:, :], preferred_element_type=jnp.float32)
        s_max = jnp.max(scores, axis=-1, keepdims=True)
        p = jnp.exp(scores - s_max)
        probs = p / jnp.sum(p, axis=-1, keepdims=True)

        e0 = idx_ref[:, 0:1]
        e1 = idx_ref[:, 1:2]
        iota = lax.broadcasted_iota(jnp.int32, (n_tok, n_exp), 1)
        g0 = jnp.sum(jnp.where(iota == e0, probs, 0.0), axis=1, keepdims=True)
        g1 = jnp.sum(jnp.where(iota == e1, probs, 0.0), axis=1, keepdims=True)
        gs = g0 + g1
        w0 = g0 / gs
        w1 = g1 / gs

        def contrib(pair_ref, origin):
            acc = None
            for le in range(E_PER):
                ge = origin * E_PER + le
                m = (w0 * (e0 == ge).astype(jnp.float32)
                     + w1 * (e1 == ge).astype(jnp.float32))
                y = jnp.dot(xv_bf, pair_ref[le],
                            preferred_element_type=jnp.float32)
                t = y * m
                acc = t if acc is None else acc + t
            return acc

        out_ref[:, :] = contrib(own_buf, my_pos)

        for hp in range(max(R_HOPS, L_HOPS)):
            if hp < R_HOPS:
                recv_wait(bufR.at[hp], recvR_sems.at[hp])
                if hp + 1 < R_HOPS:
                    sends.append(send(bufR.at[hp], bufR.at[hp + 1],
                                      sendR_sems.at[hp + 1],
                                      recvR_sems.at[hp + 1], right))
            if hp < L_HOPS:
                recv_wait(bufL.at[hp], recvL_sems.at[hp])
                if hp + 1 < L_HOPS:
                    sends.append(send(bufL.at[hp], bufL.at[hp + 1],
                                      sendL_sems.at[hp + 1],
                                      recvL_sems.at[hp + 1], left))
            if hp < R_HOPS:
                origin = lax.rem(my_pos - hp - 1 + N_DEV, N_DEV)
                out_ref[:, :] = out_ref[:, :] + contrib(bufR.at[hp], origin)
            if hp < L_HOPS:
                origin = lax.rem(my_pos + hp + 1, N_DEV)
                out_ref[:, :] = out_ref[:, :] + contrib(bufL.at[hp], origin)

        for s in sends:
            s.wait_send()

    return pl.pallas_call(
        body,
        out_shape=jax.ShapeDtypeStruct((n_tok, h), jnp.float32),
        in_specs=[pl.BlockSpec(memory_space=pltpu.VMEM)] * 4,
        out_specs=pl.BlockSpec(memory_space=pltpu.VMEM),
        scratch_shapes=[
            pltpu.VMEM((E_PER, d, h), jnp.bfloat16),
            pltpu.VMEM((R_HOPS, E_PER, d, h), jnp.bfloat16),
            pltpu.VMEM((L_HOPS, E_PER, d, h), jnp.bfloat16),
            pltpu.SemaphoreType.DMA((R_HOPS,)),
            pltpu.SemaphoreType.DMA((R_HOPS,)),
            pltpu.SemaphoreType.DMA((L_HOPS,)),
            pltpu.SemaphoreType.DMA((L_HOPS,)),
        ],
        compiler_params=pltpu.CompilerParams(collective_id=0),
    )(x, router_W, route_idx, expert_W)


# device time: 70467 ns/iter; 2.2823x vs baseline; 1.0284x over previous
import jax
import jax.numpy as jnp
from jax import lax
from jax.experimental import pallas as pl
from jax.experimental.pallas import tpu as pltpu

N_DEV = 32
E_PER = 2
R_HOPS = N_DEV // 2
L_HOPS = N_DEV - 1 - R_HOPS


def kernel(x, router_W, route_idx, expert_W):
    n_tok, d = x.shape
    n_exp = router_W.shape[1]
    h = expert_W.shape[2]

    def body(x_ref, rw_ref, idx_ref, ew_ref, out_ref,
             own_buf, bufR, bufL,
             sendR_sems, recvR_sems, sendL_sems, recvL_sems):
        my_pos = lax.axis_index("i")
        left = lax.rem(my_pos - 1 + N_DEV, N_DEV)
        right = lax.rem(my_pos + 1, N_DEV)

        barrier_sem = pltpu.get_barrier_semaphore()
        for nbr in (left, right):
            pl.semaphore_signal(
                barrier_sem, inc=1,
                device_id=(nbr,), device_id_type=pl.DeviceIdType.MESH,
            )
        pl.semaphore_wait(barrier_sem, 2)

        def send(src_ref, dst_ref, send_sem, recv_sem, dev):
            rdma = pltpu.make_async_remote_copy(
                src_ref=src_ref, dst_ref=dst_ref,
                send_sem=send_sem, recv_sem=recv_sem,
                device_id=(dev,), device_id_type=pl.DeviceIdType.MESH,
            )
            rdma.start()
            return rdma

        def recv_wait(dst_ref, recv_sem):
            rdma = pltpu.make_async_remote_copy(
                src_ref=dst_ref, dst_ref=dst_ref,
                send_sem=recv_sem, recv_sem=recv_sem,
                device_id=(left,), device_id_type=pl.DeviceIdType.MESH,
            )
            rdma.wait_recv()

        own_buf[:, 0:h] = ew_ref[0].astype(jnp.bfloat16)
        own_buf[:, h:2 * h] = ew_ref[1].astype(jnp.bfloat16)
        sends = [
            send(own_buf, bufR.at[0], sendR_sems.at[0], recvR_sems.at[0], right),
            send(own_buf, bufL.at[0], sendL_sems.at[0], recvL_sems.at[0], left),
        ]

        xv = x_ref[:, :]
        xv_bf = xv.astype(jnp.bfloat16)
        scores = jnp.dot(xv, rw_ref[:, :], preferred_element_type=jnp.float32)
        s_max = jnp.max(scores, axis=-1, keepdims=True)
        p = jnp.exp(scores - s_max)
        probs = p / jnp.sum(p, axis=-1, keepdims=True)

        e0 = idx_ref[:, 0:1]
        e1 = idx_ref[:, 1:2]
        iota = lax.broadcasted_iota(jnp.int32, (n_tok, n_exp), 1)
        g0 = jnp.sum(jnp.where(iota == e0, probs, 0.0), axis=1, keepdims=True)
        g1 = jnp.sum(jnp.where(iota == e1, probs, 0.0), axis=1, keepdims=True)
        gs = g0 + g1
        w0 = g0 / gs
        w1 = g1 / gs

        def contrib(pair_ref, origin):
            y = jnp.dot(xv_bf, pair_ref[...],
                        preferred_element_type=jnp.float32)
            ge0 = origin * E_PER
            ge1 = ge0 + 1
            m0 = (w0 * (e0 == ge0).astype(jnp.float32)
                  + w1 * (e1 == ge0).astype(jnp.float32))
            m1 = (w0 * (e0 == ge1).astype(jnp.float32)
                  + w1 * (e1 == ge1).astype(jnp.float32))
            return y[:, 0:h] * m0 + y[:, h:2 * h] * m1

        out_ref[:, :] = contrib(own_buf, my_pos)

        for hp in range(max(R_HOPS, L_HOPS)):
            if hp < R_HOPS:
                recv_wait(bufR.at[hp], recvR_sems.at[hp])
                if hp + 1 < R_HOPS:
                    sends.append(send(bufR.at[hp], bufR.at[hp + 1],
                                      sendR_sems.at[hp + 1],
                                      recvR_sems.at[hp + 1], right))
            if hp < L_HOPS:
                recv_wait(bufL.at[hp], recvL_sems.at[hp])
                if hp + 1 < L_HOPS:
                    sends.append(send(bufL.at[hp], bufL.at[hp + 1],
                                      sendL_sems.at[hp + 1],
                                      recvL_sems.at[hp + 1], left))
            if hp < R_HOPS:
                origin = lax.rem(my_pos - hp - 1 + N_DEV, N_DEV)
                out_ref[:, :] = out_ref[:, :] + contrib(bufR.at[hp], origin)
            if hp < L_HOPS:
                origin = lax.rem(my_pos + hp + 1, N_DEV)
                out_ref[:, :] = out_ref[:, :] + contrib(bufL.at[hp], origin)

        for s in sends:
            s.wait_send()

    return pl.pallas_call(
        body,
        out_shape=jax.ShapeDtypeStruct((n_tok, h), jnp.float32),
        in_specs=[pl.BlockSpec(memory_space=pltpu.VMEM)] * 4,
        out_specs=pl.BlockSpec(memory_space=pltpu.VMEM),
        scratch_shapes=[
            pltpu.VMEM((d, E_PER * h), jnp.bfloat16),
            pltpu.VMEM((R_HOPS, d, E_PER * h), jnp.bfloat16),
            pltpu.VMEM((L_HOPS, d, E_PER * h), jnp.bfloat16),
            pltpu.SemaphoreType.DMA((R_HOPS,)),
            pltpu.SemaphoreType.DMA((R_HOPS,)),
            pltpu.SemaphoreType.DMA((L_HOPS,)),
            pltpu.SemaphoreType.DMA((L_HOPS,)),
        ],
        compiler_params=pltpu.CompilerParams(collective_id=0),
    )(x, router_W, route_idx, expert_W)


# device time: 57897 ns/iter; 2.7779x vs baseline; 1.2171x over previous
import jax
import jax.numpy as jnp
from jax import lax
from jax.experimental import pallas as pl
from jax.experimental.pallas import tpu as pltpu

N_DEV = 32
E_PER = 2
R_HOPS = N_DEV // 2
L_HOPS = N_DEV - 1 - R_HOPS

PERM = [1, 2, 5, 6, 14, 13, 10, 9, 17, 18, 21, 22, 30, 29, 26, 25,
        24, 27, 28, 31, 23, 20, 19, 16, 8, 11, 12, 15, 7, 4, 3, 0]
INV = [0] * N_DEV
for _k, _m in enumerate(PERM):
    INV[_m] = _k


def kernel(x, router_W, route_idx, expert_W):
    n_tok, d = x.shape
    n_exp = router_W.shape[1]
    h = expert_W.shape[2]

    def body(x_ref, rw_ref, idx_ref, ew_ref, out_ref,
             own_buf, bufR, bufL,
             sendR_sems, recvR_sems, sendL_sems, recvL_sems):
        my_pos = lax.axis_index("i")

        iota_row = lax.broadcasted_iota(jnp.int32, (1, N_DEV), 1)
        perm_row = jnp.zeros((1, N_DEV), jnp.int32)
        inv_row = jnp.zeros((1, N_DEV), jnp.int32)
        for k in range(N_DEV):
            perm_row = jnp.where(iota_row == k, PERM[k], perm_row)
            inv_row = jnp.where(iota_row == k, INV[k], inv_row)

        def lookup(row, idx):
            return jnp.sum(jnp.where(iota_row == idx, row, 0))

        r = lookup(inv_row, my_pos)
        right = lookup(perm_row, lax.rem(r + 1, N_DEV))
        left = lookup(perm_row, lax.rem(r - 1 + N_DEV, N_DEV))

        barrier_sem = pltpu.get_barrier_semaphore()
        for nbr in (left, right):
            pl.semaphore_signal(
                barrier_sem, inc=1,
                device_id=(nbr,), device_id_type=pl.DeviceIdType.MESH,
            )
        pl.semaphore_wait(barrier_sem, 2)

        def send(src_ref, dst_ref, send_sem, recv_sem, dev):
            rdma = pltpu.make_async_remote_copy(
                src_ref=src_ref, dst_ref=dst_ref,
                send_sem=send_sem, recv_sem=recv_sem,
                device_id=(dev,), device_id_type=pl.DeviceIdType.MESH,
            )
            rdma.start()
            return rdma

        def recv_wait(dst_ref, recv_sem):
            rdma = pltpu.make_async_remote_copy(
                src_ref=dst_ref, dst_ref=dst_ref,
                send_sem=recv_sem, recv_sem=recv_sem,
                device_id=(left,), device_id_type=pl.DeviceIdType.MESH,
            )
            rdma.wait_recv()

        own_buf[:, 0:h] = ew_ref[0].astype(jnp.bfloat16)
        own_buf[:, h:2 * h] = ew_ref[1].astype(jnp.bfloat16)
        sends = [
            send(own_buf, bufR.at[0], sendR_sems.at[0], recvR_sems.at[0], right),
            send(own_buf, bufL.at[0], sendL_sems.at[0], recvL_sems.at[0], left),
        ]

        xv = x_ref[:, :]
        xv_bf = xv.astype(jnp.bfloat16)
        scores = jnp.dot(xv, rw_ref[:, :], preferred_element_type=jnp.float32)
        s_max = jnp.max(scores, axis=-1, keepdims=True)
        p = jnp.exp(scores - s_max)
        probs = p / jnp.sum(p, axis=-1, keepdims=True)

        e0 = idx_ref[:, 0:1]
        e1 = idx_ref[:, 1:2]
        iota = lax.broadcasted_iota(jnp.int32, (n_tok, n_exp), 1)
        g0 = jnp.sum(jnp.where(iota == e0, probs, 0.0), axis=1, keepdims=True)
        g1 = jnp.sum(jnp.where(iota == e1, probs, 0.0), axis=1, keepdims=True)
        gs = g0 + g1
        w0 = g0 / gs
        w1 = g1 / gs

        def contrib(pair_ref, origin):
            y = jnp.dot(xv_bf, pair_ref[...],
                        preferred_element_type=jnp.float32)
            ge0 = origin * E_PER
            ge1 = ge0 + 1
            m0 = (w0 * (e0 == ge0).astype(jnp.float32)
                  + w1 * (e1 == ge0).astype(jnp.float32))
            m1 = (w0 * (e0 == ge1).astype(jnp.float32)
                  + w1 * (e1 == ge1).astype(jnp.float32))
            return y[:, 0:h] * m0 + y[:, h:2 * h] * m1

        out_ref[:, :] = contrib(own_buf, my_pos)

        for hp in range(max(R_HOPS, L_HOPS)):
            if hp < R_HOPS:
                recv_wait(bufR.at[hp], recvR_sems.at[hp])
                if hp + 1 < R_HOPS:
                    sends.append(send(bufR.at[hp], bufR.at[hp + 1],
                                      sendR_sems.at[hp + 1],
                                      recvR_sems.at[hp + 1], right))
            if hp < L_HOPS:
                recv_wait(bufL.at[hp], recvL_sems.at[hp])
                if hp + 1 < L_HOPS:
                    sends.append(send(bufL.at[hp], bufL.at[hp + 1],
                                      sendL_sems.at[hp + 1],
                                      recvL_sems.at[hp + 1], left))
            if hp < R_HOPS:
                origin = lookup(perm_row, lax.rem(r - hp - 1 + N_DEV, N_DEV))
                out_ref[:, :] = out_ref[:, :] + contrib(bufR.at[hp], origin)
            if hp < L_HOPS:
                origin = lookup(perm_row, lax.rem(r + hp + 1, N_DEV))
                out_ref[:, :] = out_ref[:, :] + contrib(bufL.at[hp], origin)

        for s in sends:
            s.wait_send()

    return pl.pallas_call(
        body,
        out_shape=jax.ShapeDtypeStruct((n_tok, h), jnp.float32),
        in_specs=[pl.BlockSpec(memory_space=pltpu.VMEM)] * 4,
        out_specs=pl.BlockSpec(memory_space=pltpu.VMEM),
        scratch_shapes=[
            pltpu.VMEM((d, E_PER * h), jnp.bfloat16),
            pltpu.VMEM((R_HOPS, d, E_PER * h), jnp.bfloat16),
            pltpu.VMEM((L_HOPS, d, E_PER * h), jnp.bfloat16),
            pltpu.SemaphoreType.DMA((R_HOPS,)),
            pltpu.SemaphoreType.DMA((R_HOPS,)),
            pltpu.SemaphoreType.DMA((L_HOPS,)),
            pltpu.SemaphoreType.DMA((L_HOPS,)),
        ],
        compiler_params=pltpu.CompilerParams(collective_id=0),
    )(x, router_W, route_idx, expert_W)
